# baseline (speedup 1.0000x reference)
import functools

import jax
import jax.numpy as jnp
from jax import lax
from jax.experimental import pallas as pl
from jax.experimental.pallas import tpu as pltpu
from jax.experimental.pallas import tpu_sc as plsc

_LANES = 16


def _rsqrt(x):
    i = lax.bitcast_convert_type(x, jnp.int32)
    i = jnp.int32(0x5F3759DF) - lax.shift_right_logical(i, jnp.int32(1))
    y = lax.bitcast_convert_type(i, jnp.float32)
    for _ in range(3):
        y = y * (jnp.float32(1.5) - jnp.float32(0.5) * x * y * y)
    return y


def _lane_sum(v):
    lane = lax.iota(jnp.int32, _LANES)
    for k in (8, 4, 2, 1):
        v = v + v.at[lane ^ k].get(mode="promise_in_bounds")
    return v


def _make_kernel(N, S, D, CH):
    info = plsc.get_sparse_core_info()
    NW = info.num_cores * info.num_subcores
    n_per_w = N // NW
    n_chunks = n_per_w // CH
    JV = D // _LANES

    NB = 4
    assert n_chunks % NB == 0 and n_chunks >= 2 * NB

    mesh = plsc.VectorSubcoreMesh(core_axis_name="c", subcore_axis_name="s")

    @functools.partial(
        pl.kernel,
        out_type=jax.ShapeDtypeStruct((N, D), jnp.float32),
        mesh=mesh,
        scratch_types=[
            pltpu.VMEM((n_per_w,), jnp.int32),
            pltpu.VMEM((NB, CH, D), jnp.float32),
            pltpu.VMEM((CH, _LANES), jnp.float32),
            pltpu.VMEM((CH, _LANES), jnp.float32),
            pltpu.VMEM((D,), jnp.float32),
            pltpu.VMEM((D,), jnp.float32),
            pltpu.SemaphoreType.DMA((NB,)),
        ],
    )
    def k(x_hbm, w_hbm, pos_hbm, gamma_hbm, beta_hbm, out_hbm,
          idx_v, rows_v, a_v, m_v, gamma_v, beta_v, sem_arr):
        wid = lax.axis_index("c") * info.num_subcores + lax.axis_index("s")
        base = wid * n_per_w
        s0 = lax.rem(base, S)

        pltpu.sync_copy(x_hbm.at[pl.ds(base, n_per_w)], idx_v)
        pltpu.sync_copy(gamma_hbm, gamma_v)
        pltpu.sync_copy(beta_hbm, beta_v)

        def pos_start(c, p):
            pltpu.async_copy(pos_hbm.at[pl.ds(s0 + c * CH, CH)],
                             rows_v.at[p], sem_arr.at[p])

        def gather_start(c, p):
            pltpu.async_copy(w_hbm.at[idx_v.at[pl.ds(c * CH, CH)]],
                             rows_v.at[p], sem_arr.at[p], add=True)

        def store_start(c, p):
            pltpu.async_copy(rows_v.at[p], out_hbm.at[pl.ds(base + c * CH, CH)],
                             sem_arr.at[p])

        def dma_wait(p):
            pltpu.make_async_copy(pos_hbm.at[pl.ds(0, CH)],
                                  rows_v.at[p], sem_arr.at[p]).wait()

        def compute(p):
            rows_b = rows_v.at[p]

            @plsc.parallel_loop(0, CH, 1, unroll=2)
            def stats_body(r):
                acc_s = [jnp.zeros((_LANES,), jnp.float32) for _ in range(4)]
                acc_q = [jnp.zeros((_LANES,), jnp.float32) for _ in range(4)]
                for j in range(JV):
                    v = rows_b[r, pl.ds(j * _LANES, _LANES)]
                    acc_s[j % 4] = acc_s[j % 4] + v
                    acc_q[j % 4] = acc_q[j % 4] + v * v
                tot_s = (acc_s[0] + acc_s[1]) + (acc_s[2] + acc_s[3])
                tot_q = (acc_q[0] + acc_q[1]) + (acc_q[2] + acc_q[3])
                mvec = _lane_sum(tot_s) * jnp.float32(1.0 / D)
                msq = _lane_sum(tot_q) * jnp.float32(1.0 / D)
                var = msq - mvec * mvec
                rsig = _rsqrt(var + jnp.float32(1e-5))
                a_v[r, :] = rsig
                m_v[r, :] = mvec * rsig

            GW = 16
            for jg in range(JV // GW):
                gs = [gamma_v[pl.ds((jg * GW + t) * _LANES, _LANES)]
                      for t in range(GW)]
                bs = [beta_v[pl.ds((jg * GW + t) * _LANES, _LANES)]
                      for t in range(GW)]

                @plsc.parallel_loop(0, CH, 1, unroll=2)
                def apply_body(r, jg=jg, gs=gs, bs=bs):
                    a = a_v[r, :]
                    m = m_v[r, :]
                    for t in range(GW):
                        j = jg * GW + t
                        v = rows_b[r, pl.ds(j * _LANES, _LANES)]
                        rows_b[r, pl.ds(j * _LANES, _LANES)] = (
                            (v * a - m) * gs[t] + bs[t])

        pos_start(0, 0)
        pos_start(1, 1)
        dma_wait(0)
        gather_start(0, 0)

        def chunk_iter(c, _):
            p = lax.bitwise_and(c, NB - 1)
            p1 = lax.bitwise_and(c + 1, NB - 1)
            p2 = lax.bitwise_and(c + 2, NB - 1)
            dma_wait(p)

            @pl.when(c >= 2)
            def _():
                dma_wait(p2)

            @pl.when(c + 2 < n_chunks)
            def _():
                pos_start(c + 2, p2)

            @pl.when(c + 1 < n_chunks)
            def _():
                dma_wait(p1)
                gather_start(c + 1, p1)

            compute(p)
            store_start(c, p)
            return 0

        lax.fori_loop(0, n_chunks, chunk_iter, 0)
        dma_wait((n_chunks - 2) % NB)
        dma_wait((n_chunks - 1) % NB)

    return k


def kernel(x, W, pos, gamma, beta):
    B, S = x.shape
    V, D = W.shape
    N = B * S
    x_flat = x.reshape(N).astype(jnp.int32)
    pos_s = pos[0, :S, :]
    k = _make_kernel(N, S, D, CH=16)
    out = k(x_flat, W, pos_s, gamma, beta)
    return out.reshape(B, S, D)


# speedup vs baseline: 1.0197x; 1.0197x over previous
import functools

import jax
import jax.numpy as jnp
from jax import lax
from jax.experimental import pallas as pl
from jax.experimental.pallas import tpu as pltpu
from jax.experimental.pallas import tpu_sc as plsc

_LANES = 16


def _rsqrt(x):
    i = lax.bitcast_convert_type(x, jnp.int32)
    i = jnp.int32(0x5F3759DF) - lax.shift_right_logical(i, jnp.int32(1))
    y = lax.bitcast_convert_type(i, jnp.float32)
    for _ in range(2):
        y = y * (jnp.float32(1.5) - jnp.float32(0.5) * x * y * y)
    return y


def _lane_sum(v):
    lane = lax.iota(jnp.int32, _LANES)
    for k in (8, 4, 2, 1):
        v = v + v.at[lane ^ k].get(mode="promise_in_bounds")
    return v


def _make_kernel(N, S, D, CH):
    info = plsc.get_sparse_core_info()
    NW = info.num_cores * info.num_subcores
    n_per_w = N // NW
    n_chunks = n_per_w // CH
    JV = D // _LANES

    NB = 4
    assert n_chunks % NB == 0 and n_chunks >= 2 * NB

    mesh = plsc.VectorSubcoreMesh(core_axis_name="c", subcore_axis_name="s")

    @functools.partial(
        pl.kernel,
        out_type=jax.ShapeDtypeStruct((N, D), jnp.float32),
        mesh=mesh,
        scratch_types=[
            pltpu.VMEM((n_per_w,), jnp.int32),
            pltpu.VMEM((NB, CH, D), jnp.float32),
            pltpu.VMEM((CH, _LANES), jnp.float32),
            pltpu.VMEM((CH, _LANES), jnp.float32),
            pltpu.VMEM((D,), jnp.float32),
            pltpu.VMEM((D,), jnp.float32),
            pltpu.SemaphoreType.DMA((NB,)),
        ],
    )
    def k(x_hbm, w_hbm, pos_hbm, gamma_hbm, beta_hbm, out_hbm,
          idx_v, rows_v, a_v, m_v, gamma_v, beta_v, sem_arr):
        wid = lax.axis_index("c") * info.num_subcores + lax.axis_index("s")
        base = wid * n_per_w
        s0 = lax.rem(base, S)

        pltpu.sync_copy(x_hbm.at[pl.ds(base, n_per_w)], idx_v)
        pltpu.sync_copy(gamma_hbm, gamma_v)
        pltpu.sync_copy(beta_hbm, beta_v)

        def pos_start(c, p):
            pltpu.async_copy(pos_hbm.at[pl.ds(s0 + c * CH, CH)],
                             rows_v.at[p], sem_arr.at[p])

        def gather_start(c, p):
            pltpu.async_copy(w_hbm.at[idx_v.at[pl.ds(c * CH, CH)]],
                             rows_v.at[p], sem_arr.at[p], add=True)

        def store_start(c, p):
            pltpu.async_copy(rows_v.at[p], out_hbm.at[pl.ds(base + c * CH, CH)],
                             sem_arr.at[p])

        def dma_wait(p):
            pltpu.make_async_copy(pos_hbm.at[pl.ds(0, CH)],
                                  rows_v.at[p], sem_arr.at[p]).wait()

        def compute(p):
            rows_b = rows_v.at[p]

            @plsc.parallel_loop(0, CH, 1, unroll=2)
            def stats_body(r):
                acc_s = [jnp.zeros((_LANES,), jnp.float32) for _ in range(4)]
                acc_q = [jnp.zeros((_LANES,), jnp.float32) for _ in range(4)]
                for j in range(JV):
                    v = rows_b[r, pl.ds(j * _LANES, _LANES)]
                    acc_s[j % 4] = acc_s[j % 4] + v
                    acc_q[j % 4] = acc_q[j % 4] + v * v
                tot_s = (acc_s[0] + acc_s[1]) + (acc_s[2] + acc_s[3])
                tot_q = (acc_q[0] + acc_q[1]) + (acc_q[2] + acc_q[3])
                mvec = _lane_sum(tot_s) * jnp.float32(1.0 / D)
                msq = _lane_sum(tot_q) * jnp.float32(1.0 / D)
                var = msq - mvec * mvec
                rsig = _rsqrt(var + jnp.float32(1e-5))
                a_v[r, :] = rsig
                m_v[r, :] = mvec * rsig

            GW = 16
            for jg in range(JV // GW):
                gs = [gamma_v[pl.ds((jg * GW + t) * _LANES, _LANES)]
                      for t in range(GW)]
                bs = [beta_v[pl.ds((jg * GW + t) * _LANES, _LANES)]
                      for t in range(GW)]

                @plsc.parallel_loop(0, CH, 1, unroll=2)
                def apply_body(r, jg=jg, gs=gs, bs=bs):
                    a = a_v[r, :]
                    m = m_v[r, :]
                    for t in range(GW):
                        j = jg * GW + t
                        v = rows_b[r, pl.ds(j * _LANES, _LANES)]
                        rows_b[r, pl.ds(j * _LANES, _LANES)] = (
                            (v * a - m) * gs[t] + bs[t])

        pos_start(0, 0)
        pos_start(1, 1)
        dma_wait(0)
        gather_start(0, 0)

        def chunk_iter(c, _):
            p = lax.bitwise_and(c, NB - 1)
            p1 = lax.bitwise_and(c + 1, NB - 1)
            p2 = lax.bitwise_and(c + 2, NB - 1)
            dma_wait(p)

            @pl.when(c >= 2)
            def _():
                dma_wait(p2)

            @pl.when(c + 2 < n_chunks)
            def _():
                pos_start(c + 2, p2)

            @pl.when(c + 1 < n_chunks)
            def _():
                dma_wait(p1)
                gather_start(c + 1, p1)

            compute(p)
            store_start(c, p)
            return 0

        lax.fori_loop(0, n_chunks, chunk_iter, 0)
        dma_wait((n_chunks - 2) % NB)
        dma_wait((n_chunks - 1) % NB)

    return k


def kernel(x, W, pos, gamma, beta):
    B, S = x.shape
    V, D = W.shape
    N = B * S
    x_flat = x.reshape(N).astype(jnp.int32)
    pos_s = pos[0, :S, :]
    k = _make_kernel(N, S, D, CH=32)
    out = k(x_flat, W, pos_s, gamma, beta)
    return out.reshape(B, S, D)
